# Initial kernel scaffold; baseline (speedup 1.0000x reference)
import jax
import jax.numpy as jnp
from jax.experimental import pallas as pl


def kernel(x):
    raise NotImplementedError("write your pallas kernel here")


# speedup vs baseline: 2.2182x; 2.2182x over previous
import functools

import jax
import jax.numpy as jnp
from jax import lax
from jax.experimental import pallas as pl
from jax.experimental.pallas import tpu as pltpu
from jax.experimental.pallas import tpu_sc as plsc

ROWS = 128
COLS = 32768
LANES = 16
NEG_BIG = float("-inf")
POS_BIG = float("inf")


def _insert_max(m, v):
    m1, m2, m3, m4 = m
    n1 = jnp.maximum(m1, v)
    t = jnp.minimum(m1, v)
    n2 = jnp.maximum(m2, t)
    t = jnp.minimum(m2, t)
    n3 = jnp.maximum(m3, t)
    t = jnp.minimum(m3, t)
    n4 = jnp.maximum(m4, t)
    return (n1, n2, n3, n4)


def _insert_min(m, v):
    m1, m2, m3, m4 = m
    n1 = jnp.minimum(m1, v)
    t = jnp.maximum(m1, v)
    n2 = jnp.minimum(m2, t)
    t = jnp.maximum(m2, t)
    n3 = jnp.minimum(m3, t)
    t = jnp.maximum(m3, t)
    n4 = jnp.minimum(m4, t)
    return (n1, n2, n3, n4)


def _make_kernel():
    info = plsc.get_sparse_core_info()
    nc, ns = info.num_cores, info.num_subcores
    nw = nc * ns
    rows_per_w = ROWS // nw
    n_iters = COLS // (2 * LANES)
    mesh = plsc.VectorSubcoreMesh(core_axis_name="c", subcore_axis_name="s")

    @functools.partial(
        pl.kernel,
        mesh=mesh,
        out_type=jax.ShapeDtypeStruct((ROWS, LANES), jnp.float32),
        scratch_types=[
            pltpu.VMEM((COLS,), jnp.float32),
            pltpu.VMEM((COLS,), jnp.float32),
            pltpu.VMEM((LANES,), jnp.float32),
            pltpu.SemaphoreType.DMA,
            pltpu.SemaphoreType.DMA,
        ],
        compiler_params=pltpu.CompilerParams(needs_layout_passes=False),
    )
    def topk_sc(x_hbm, out_hbm, buf0, buf1, outv, sem0, sem1):
        wid = lax.axis_index("s") * nc + lax.axis_index("c")
        row0 = wid * rows_per_w
        iota = lax.iota(jnp.int32, LANES)
        bufs = (buf0, buf1)
        sems = (sem0, sem1)

        handle = pltpu.async_copy(x_hbm.at[row0], bufs[0], sems[0])
        for r in range(rows_per_w):
            cur = bufs[r % 2]
            if r + 1 < rows_per_w:
                nxt_handle = pltpu.async_copy(
                    x_hbm.at[row0 + r + 1], bufs[(r + 1) % 2], sems[(r + 1) % 2]
                )
            handle.wait()

            def body(i, carry, cur=cur):
                amax = carry[0:4]
                amin = carry[4:8]
                bmax = carry[8:12]
                bmin = carry[12:16]
                base = i * (2 * LANES)
                v0 = cur[pl.ds(base, LANES)]
                v1 = cur[pl.ds(base + LANES, LANES)]
                amax = _insert_max(amax, v0)
                amin = _insert_min(amin, v0)
                bmax = _insert_max(bmax, v1)
                bmin = _insert_min(bmin, v1)
                return amax + amin + bmax + bmin

            neg = jnp.full((LANES,), NEG_BIG, jnp.float32)
            pos = jnp.full((LANES,), POS_BIG, jnp.float32)
            init = (neg,) * 4 + (pos,) * 4 + (neg,) * 4 + (pos,) * 4
            fin = lax.fori_loop(0, n_iters, body, init, unroll=4)
            amax, amin = fin[0:4], fin[4:8]
            bmax, bmin = fin[8:12], fin[12:16]

            hi = [jnp.maximum(amax[i], bmax[3 - i]) for i in range(4)]
            lo = [jnp.minimum(amin[i], bmin[3 - i]) for i in range(4)]

            hi_s = [jnp.sort(v) for v in hi]
            lo_s = [jnp.sort(v) for v in lo]

            l1 = l2 = l3 = l4 = jnp.full((LANES,), NEG_BIG, jnp.float32)
            s1 = s2 = s3 = s4 = jnp.full((LANES,), POS_BIG, jnp.float32)
            for j in range(4):
                for t in range(4):
                    v = jnp.full((LANES,), hi_s[j][15 - t], jnp.float32)
                    (l1, l2, l3, l4) = _insert_max((l1, l2, l3, l4), v)
                    w = jnp.full((LANES,), lo_s[j][t], jnp.float32)
                    (s1, s2, s3, s4) = _insert_min((s1, s2, s3, s4), w)

            res = jnp.where(iota == 0, l1, jnp.float32(0.0))
            res = jnp.where(iota == 1, l2, res)
            res = jnp.where(iota == 2, l3, res)
            res = jnp.where(iota == 3, l4, res)
            res = jnp.where(iota == 4, s1, res)
            res = jnp.where(iota == 5, s2, res)
            res = jnp.where(iota == 6, s3, res)
            res = jnp.where(iota == 7, s4, res)
            outv[...] = res
            pltpu.sync_copy(outv, out_hbm.at[row0 + r])
            if r + 1 < rows_per_w:
                handle = nxt_handle

    return topk_sc


_topk = _make_kernel()


@jax.jit
def kernel(x):
    res = _topk(x)
    return (res[:, 0:4], res[:, 4:8])


# speedup vs baseline: 2.6145x; 1.1787x over previous
import functools

import jax
import jax.numpy as jnp
from jax import lax
from jax.experimental import pallas as pl
from jax.experimental.pallas import tpu as pltpu
from jax.experimental.pallas import tpu_sc as plsc

ROWS = 128
COLS = 32768
LANES = 16
GROUP = 4 * LANES
NEG_BIG = float("-inf")
POS_BIG = float("inf")


def _insert_max4(m, v):
    m1, m2, m3, m4 = m
    n1 = jnp.maximum(m1, v)
    t = jnp.minimum(m1, v)
    n2 = jnp.maximum(m2, t)
    t = jnp.minimum(m2, t)
    n3 = jnp.maximum(m3, t)
    t = jnp.minimum(m3, t)
    n4 = jnp.maximum(m4, t)
    return (n1, n2, n3, n4)


def _insert_min4(m, v):
    m1, m2, m3, m4 = m
    n1 = jnp.minimum(m1, v)
    t = jnp.maximum(m1, v)
    n2 = jnp.minimum(m2, t)
    t = jnp.maximum(m2, t)
    n3 = jnp.minimum(m3, t)
    t = jnp.maximum(m3, t)
    n4 = jnp.minimum(m4, t)
    return (n1, n2, n3, n4)


def _insert_max2(m, v):
    m1, m2 = m
    n1 = jnp.maximum(m1, v)
    t = jnp.minimum(m1, v)
    n2 = jnp.maximum(m2, t)
    return (n1, n2)


def _insert_min2(m, v):
    m1, m2 = m
    n1 = jnp.minimum(m1, v)
    t = jnp.maximum(m1, v)
    n2 = jnp.minimum(m2, t)
    return (n1, n2)


def _sort4(a, b, c, d):
    a1 = jnp.maximum(a, b)
    a2 = jnp.minimum(a, b)
    b1 = jnp.maximum(c, d)
    b2 = jnp.minimum(c, d)
    w1 = jnp.maximum(a1, b1)
    t1 = jnp.minimum(a1, b1)
    w4 = jnp.minimum(a2, b2)
    t2 = jnp.maximum(a2, b2)
    w2 = jnp.maximum(t1, t2)
    w3 = jnp.minimum(t1, t2)
    return w1, w2, w3, w4


def _make_kernel():
    info = plsc.get_sparse_core_info()
    nc, ns = info.num_cores, info.num_subcores
    nw = nc * ns
    rows_per_w = ROWS // nw
    n_iters = COLS // GROUP
    mesh = plsc.VectorSubcoreMesh(core_axis_name="c", subcore_axis_name="s")

    @functools.partial(
        pl.kernel,
        mesh=mesh,
        out_type=jax.ShapeDtypeStruct((ROWS, LANES), jnp.float32),
        scratch_types=[
            pltpu.VMEM((COLS,), jnp.float32),
            pltpu.VMEM((COLS,), jnp.float32),
            pltpu.VMEM((LANES,), jnp.float32),
            pltpu.SemaphoreType.DMA,
            pltpu.SemaphoreType.DMA,
        ],
        compiler_params=pltpu.CompilerParams(needs_layout_passes=False),
    )
    def topk_sc(x_hbm, out_hbm, buf0, buf1, outv, sem0, sem1):
        wid = lax.axis_index("s") * nc + lax.axis_index("c")
        row0 = wid * rows_per_w
        iota = lax.iota(jnp.int32, LANES)
        bufs = (buf0, buf1)
        sems = (sem0, sem1)

        handle = pltpu.async_copy(x_hbm.at[row0], bufs[0], sems[0])
        for r in range(rows_per_w):
            cur = bufs[r % 2]
            if r + 1 < rows_per_w:
                nxt_handle = pltpu.async_copy(
                    x_hbm.at[row0 + r + 1], bufs[(r + 1) % 2], sems[(r + 1) % 2]
                )
            handle.wait()

            def body(i, carry, cur=cur):
                (m1, m2, m3, m4, p1, p2, q1, r1,
                 u1, u2, u3, u4, s1, s2, e1, f1) = carry
                base = i * GROUP
                a = cur[pl.ds(base, LANES)]
                b = cur[pl.ds(base + LANES, LANES)]
                c = cur[pl.ds(base + 2 * LANES, LANES)]
                d = cur[pl.ds(base + 3 * LANES, LANES)]
                w1, w2, w3, w4 = _sort4(a, b, c, d)
                (m1, m2, m3, m4) = _insert_max4((m1, m2, m3, m4), w1)
                (p1, p2) = _insert_max2((p1, p2), w2)
                q1 = jnp.maximum(q1, w3)
                r1 = jnp.maximum(r1, w4)
                (u1, u2, u3, u4) = _insert_min4((u1, u2, u3, u4), w4)
                (s1, s2) = _insert_min2((s1, s2), w3)
                e1 = jnp.minimum(e1, w2)
                f1 = jnp.minimum(f1, w1)
                return (m1, m2, m3, m4, p1, p2, q1, r1,
                        u1, u2, u3, u4, s1, s2, e1, f1)

            neg = jnp.full((LANES,), NEG_BIG, jnp.float32)
            pos = jnp.full((LANES,), POS_BIG, jnp.float32)
            init = (neg,) * 8 + (pos,) * 8
            fin = lax.fori_loop(0, n_iters, body, init, unroll=4)
            max_c = fin[0:8]
            min_c = fin[8:16]

            hi_s = [jnp.sort(v) for v in max_c]
            lo_s = [jnp.sort(v) for v in min_c]

            l1 = l2 = l3 = l4 = jnp.full((LANES,), NEG_BIG, jnp.float32)
            s1 = s2 = s3 = s4 = jnp.full((LANES,), POS_BIG, jnp.float32)
            for j in range(8):
                for t in range(4):
                    v = jnp.full((LANES,), hi_s[j][15 - t], jnp.float32)
                    (l1, l2, l3, l4) = _insert_max4((l1, l2, l3, l4), v)
                    w = jnp.full((LANES,), lo_s[j][t], jnp.float32)
                    (s1, s2, s3, s4) = _insert_min4((s1, s2, s3, s4), w)

            res = jnp.where(iota == 0, l1, jnp.float32(0.0))
            res = jnp.where(iota == 1, l2, res)
            res = jnp.where(iota == 2, l3, res)
            res = jnp.where(iota == 3, l4, res)
            res = jnp.where(iota == 4, s1, res)
            res = jnp.where(iota == 5, s2, res)
            res = jnp.where(iota == 6, s3, res)
            res = jnp.where(iota == 7, s4, res)
            outv[...] = res
            pltpu.sync_copy(outv, out_hbm.at[row0 + r])
            if r + 1 < rows_per_w:
                handle = nxt_handle

    return topk_sc


_topk = _make_kernel()


@jax.jit
def kernel(x):
    res = _topk(x)
    return (res[:, 0:4], res[:, 4:8])
